# baseline (speedup 1.0000x reference)
import functools

import jax
import jax.numpy as jnp
from jax import lax
from jax.experimental import pallas as pl
from jax.experimental.pallas import tpu as pltpu
from jax.experimental.pallas import tpu_sc as plsc

_D = 128
_CH = 128
_BLK = 4096
_INV_LN2 = 1.4426950408889634


def _gather3(user2d, pos2d, neg2d, user_table, item_table, batch):
    info = plsc.get_sparse_core_info()
    nw = info.num_cores * info.num_subcores
    rows_per_w = batch // nw
    cpg = rows_per_w // _CH
    ntask = 3 * cpg
    mesh = plsc.VectorSubcoreMesh(core_axis_name="c", subcore_axis_name="s")

    @functools.partial(
        pl.kernel,
        mesh=mesh,
        out_type=(jax.ShapeDtypeStruct((batch, _D), jnp.float32),) * 3,
        scratch_types=[
            pltpu.VMEM((ntask, _CH), jnp.int32),
            pltpu.VMEM((6, _CH, _D), jnp.float32),
            pltpu.SemaphoreType.DMA,
            pltpu.SemaphoreType.DMA,
            pltpu.SemaphoreType.DMA,
        ],
    )
    def k(user_h, pos_h, neg_h, ut_h, it_h, eu_h, ep_h, en_h,
          idx_v, rows_v, gsem, ssem, isem):
        wid = lax.axis_index("s") * info.num_cores + lax.axis_index("c")
        rbase = wid * rows_per_w
        irow0 = wid * cpg
        ic = [pltpu.async_copy(src.at[pl.ds(irow0, cpg)],
                               idx_v.at[pl.ds(off * cpg, cpg)], isem)
              for off, src in enumerate((user_h, pos_h, neg_h))]
        for c in ic:
            c.wait()

        streams = ((ut_h, eu_h), (it_h, ep_h), (it_h, en_h))

        def body(i, _):
            par3 = (i % 2) * 3
            gs = [pltpu.async_copy(tbl.at[idx_v.at[j * cpg + i]],
                                   rows_v.at[par3 + j], gsem)
                  for j, (tbl, _o) in enumerate(streams)]

            @pl.when(i > 0)
            def _drain_prev():
                p3 = 3 - par3
                for j, (_t, out) in enumerate(streams):
                    pltpu.make_async_copy(
                        rows_v.at[p3 + j],
                        out.at[pl.ds(rbase + (i - 1) * _CH, _CH)],
                        ssem).wait()

            for g in gs:
                g.wait()
            for j, (_t, out) in enumerate(streams):
                pltpu.async_copy(rows_v.at[par3 + j],
                                 out.at[pl.ds(rbase + i * _CH, _CH)], ssem)
            return _

        lax.fori_loop(0, cpg, body, None)
        lpar3 = ((cpg - 1) % 2) * 3
        for j, (_t, out) in enumerate(streams):
            pltpu.make_async_copy(
                rows_v.at[lpar3 + j],
                out.at[pl.ds(rbase + (cpg - 1) * _CH, _CH)], ssem).wait()

    return k(user2d, pos2d, neg2d, user_table, item_table)


_RT = (((1,), (1,)), ((), ()))


def _mlp_body(eu_ref, ep_ref, en_ref, w0u_ref, w0i_ref, b0_ref,
              w1_ref, b1_ref, w2_ref, b2_ref, wp_ref, bp_ref,
              pp_ref, pn_ref, loss_ref):
    eu = eu_ref[...]
    ep = ep_ref[...]
    en = en_ref[...]
    aT = lax.dot_general(w0u_ref[...], eu, _RT,
                         preferred_element_type=jnp.float32)
    b0 = b0_ref[...]
    h0p = jnp.maximum(
        aT + lax.dot_general(w0i_ref[...], ep, _RT,
                             preferred_element_type=jnp.float32) + b0, 0.0)
    h0n = jnp.maximum(
        aT + lax.dot_general(w0i_ref[...], en, _RT,
                             preferred_element_type=jnp.float32) + b0, 0.0)
    w1 = w1_ref[...]
    b1 = b1_ref[...]
    h1p = jnp.maximum(
        jnp.dot(w1, h0p, preferred_element_type=jnp.float32) + b1, 0.0)
    h1n = jnp.maximum(
        jnp.dot(w1, h0n, preferred_element_type=jnp.float32) + b1, 0.0)
    w2 = w2_ref[...]
    b2 = b2_ref[...]
    h2p = jnp.maximum(
        jnp.dot(w2, h1p, preferred_element_type=jnp.float32) + b2, 0.0)
    h2n = jnp.maximum(
        jnp.dot(w2, h1n, preferred_element_type=jnp.float32) + b2, 0.0)
    wp = wp_ref[...]
    bp = bp_ref[0, 0]
    pp = jnp.sum(h2p * wp, axis=0, keepdims=True) + bp
    pn = jnp.sum(h2n * wp, axis=0, keepdims=True) + bp
    pp_ref[...] = pp[None]
    pn_ref[...] = pn[None]
    d = pp - pn
    l2 = -(jnp.maximum(-d, 0.0)
           + jnp.log(1.0 + jnp.exp(-jnp.abs(d)))) * _INV_LN2
    blk_loss = -jnp.sum(l2)

    @pl.when(pl.program_id(0) == 0)
    def _init():
        loss_ref[0, 0] = jnp.zeros((), jnp.float32)

    loss_ref[0, 0] += blk_loss


def _mlp(eu, ep, en, w0u, w0i, b0r, w1p, b1p, w2p, b2p, wpp, bp11, batch):
    n_blk = batch // _BLK
    row_spec = pl.BlockSpec((_BLK, _D), lambda i: (i, 0))
    w_spec = pl.BlockSpec((_D, _D), lambda i: (0, 0))
    c_spec = pl.BlockSpec((_D, 1), lambda i: (0, 0))
    return pl.pallas_call(
        _mlp_body,
        grid=(n_blk,),
        in_specs=[row_spec, row_spec, row_spec,
                  w_spec, w_spec, c_spec,
                  w_spec, c_spec,
                  w_spec, c_spec,
                  c_spec,
                  pl.BlockSpec(memory_space=pltpu.SMEM)],
        out_specs=[pl.BlockSpec((1, 1, _BLK), lambda i: (i, 0, 0)),
                   pl.BlockSpec((1, 1, _BLK), lambda i: (i, 0, 0)),
                   pl.BlockSpec(memory_space=pltpu.SMEM)],
        out_shape=[jax.ShapeDtypeStruct((n_blk, 1, _BLK), jnp.float32),
                   jax.ShapeDtypeStruct((n_blk, 1, _BLK), jnp.float32),
                   jax.ShapeDtypeStruct((1, 1), jnp.float32)],
    )(eu, ep, en, w0u, w0i, b0r, w1p, b1p, w2p, b2p, wpp, bp11)


_NCHUNK = 1


def kernel(user, pos_item, neg_item, user_table, item_table,
           W0, b0, W1, b1, W2, b2, Wp, bp):
    batch = user.shape[0]
    user2d = user.astype(jnp.int32).reshape(batch // _CH, _CH)
    pos2d = pos_item.astype(jnp.int32).reshape(batch // _CH, _CH)
    neg2d = neg_item.astype(jnp.int32).reshape(batch // _CH, _CH)

    w0u = W0[:_D].T
    w0i = W0[_D:].T
    b0r = b0.reshape(_D, 1)
    w1p = jnp.zeros((_D, _D), jnp.float32).at[:64, :].set(W1.T)
    b1p = jnp.zeros((_D, 1), jnp.float32).at[:64, 0].set(b1)
    w2p = jnp.zeros((_D, _D), jnp.float32).at[:32, :64].set(W2.T)
    b2p = jnp.zeros((_D, 1), jnp.float32).at[:32, 0].set(b2)
    wpp = jnp.zeros((_D, 1), jnp.float32).at[:32, 0].set(Wp[:, 0])
    bp11 = bp.reshape(1, 1)

    rows = batch // _NCHUNK
    irows = rows // _CH
    pps, pns, losses = [], [], []
    for c in range(_NCHUNK):
        sl = slice(c * irows, (c + 1) * irows)
        eu, ep, en = _gather3(user2d[sl], pos2d[sl], neg2d[sl],
                              user_table, item_table, rows)
        pp, pn, ls = _mlp(eu, ep, en, w0u, w0i, b0r, w1p, b1p, w2p, b2p,
                          wpp, bp11, rows)
        pps.append(pp.reshape(rows))
        pns.append(pn.reshape(rows))
        losses.append(ls)
    pp = jnp.concatenate(pps) if _NCHUNK > 1 else pps[0]
    pn = jnp.concatenate(pns) if _NCHUNK > 1 else pns[0]
    loss = sum(losses[1:], losses[0])
    return pp, pn, loss.reshape(())


# speedup vs baseline: 1.0256x; 1.0256x over previous
import functools

import jax
import jax.numpy as jnp
from jax import lax
from jax.experimental import pallas as pl
from jax.experimental.pallas import tpu as pltpu
from jax.experimental.pallas import tpu_sc as plsc

_D = 128
_CH = 128
_BLK = 4096
_INV_LN2 = 1.4426950408889634


def _gather3(user2d, pos2d, neg2d, user_table, item_table, batch):
    info = plsc.get_sparse_core_info()
    nw = info.num_cores * info.num_subcores
    rows_per_w = batch // nw
    cpg = rows_per_w // _CH
    ntask = 3 * cpg
    mesh = plsc.VectorSubcoreMesh(core_axis_name="c", subcore_axis_name="s")

    @functools.partial(
        pl.kernel,
        mesh=mesh,
        out_type=(jax.ShapeDtypeStruct((batch, _D), jnp.float32),) * 3,
        scratch_types=[
            pltpu.VMEM((ntask, _CH), jnp.int32),
            pltpu.VMEM((4, _CH, _D), jnp.float32),
            pltpu.SemaphoreType.DMA,
            pltpu.SemaphoreType.DMA,
            pltpu.SemaphoreType.DMA,
            pltpu.SemaphoreType.DMA,
            pltpu.SemaphoreType.DMA,
            pltpu.SemaphoreType.DMA,
            pltpu.SemaphoreType.DMA,
            pltpu.SemaphoreType.DMA,
            pltpu.SemaphoreType.DMA,
        ],
    )
    def k(user_h, pos_h, neg_h, ut_h, it_h, eu_h, ep_h, en_h,
          idx_v, rows_v, g0, g1, g2, g3, s0, s1, s2, s3, isem):
        wid = lax.axis_index("s") * info.num_cores + lax.axis_index("c")
        rbase = wid * rows_per_w
        irow0 = wid * cpg
        ic = [pltpu.async_copy(src.at[pl.ds(irow0, cpg)],
                               idx_v.at[pl.ds(off * cpg, cpg)], isem)
              for off, src in enumerate((user_h, pos_h, neg_h))]
        for c in ic:
            c.wait()

        gsem = (g0, g1, g2, g3)
        ssem = (s0, s1, s2, s3)
        tasks = []
        for j, (tbl, out) in enumerate(((ut_h, eu_h), (it_h, ep_h), (it_h, en_h))):
            for c in range(cpg):
                tasks.append((j * cpg + c, tbl, out, rbase + c * _CH))

        scat = [None, None, None, None]
        q = []

        def drain_one():
            pg, pb, pout, pobase = q.pop(0)
            pg.wait()
            scat[pb] = pltpu.async_copy(
                rows_v.at[pb], pout.at[pl.ds(pobase, _CH)], ssem[pb])

        for t, (ti, tbl, out, obase) in enumerate(tasks):
            b = t % 4
            if scat[b] is not None:
                scat[b].wait()
                scat[b] = None
            q.append((pltpu.async_copy(tbl.at[idx_v.at[ti]], rows_v.at[b],
                                       gsem[b]), b, out, obase))
            if len(q) >= 3:
                drain_one()
        while q:
            drain_one()
        for sc in scat:
            if sc is not None:
                sc.wait()

    return k(user2d, pos2d, neg2d, user_table, item_table)


_RT = (((1,), (1,)), ((), ()))


def _mlp_body(eu_ref, ep_ref, en_ref, w0u_ref, w0i_ref, b0_ref,
              w1_ref, b1_ref, w2_ref, b2_ref, wp_ref, bp_ref,
              pp_ref, pn_ref, loss_ref):
    eu = eu_ref[...]
    ep = ep_ref[...]
    en = en_ref[...]
    aT = lax.dot_general(w0u_ref[...], eu, _RT,
                         preferred_element_type=jnp.float32)
    b0 = b0_ref[...]
    h0p = jnp.maximum(
        aT + lax.dot_general(w0i_ref[...], ep, _RT,
                             preferred_element_type=jnp.float32) + b0, 0.0)
    h0n = jnp.maximum(
        aT + lax.dot_general(w0i_ref[...], en, _RT,
                             preferred_element_type=jnp.float32) + b0, 0.0)
    w1 = w1_ref[...]
    b1 = b1_ref[...]
    h1p = jnp.maximum(
        jnp.dot(w1, h0p, preferred_element_type=jnp.float32) + b1, 0.0)
    h1n = jnp.maximum(
        jnp.dot(w1, h0n, preferred_element_type=jnp.float32) + b1, 0.0)
    w2 = w2_ref[...]
    b2 = b2_ref[...]
    h2p = jnp.maximum(
        jnp.dot(w2, h1p, preferred_element_type=jnp.float32) + b2, 0.0)
    h2n = jnp.maximum(
        jnp.dot(w2, h1n, preferred_element_type=jnp.float32) + b2, 0.0)
    wp = wp_ref[...]
    bp = bp_ref[0, 0]
    pp = jnp.sum(h2p * wp, axis=0, keepdims=True) + bp
    pn = jnp.sum(h2n * wp, axis=0, keepdims=True) + bp
    pp_ref[...] = pp[None]
    pn_ref[...] = pn[None]
    d = pp - pn
    l2 = -(jnp.maximum(-d, 0.0)
           + jnp.log(1.0 + jnp.exp(-jnp.abs(d)))) * _INV_LN2
    blk_loss = jnp.full((1, _D), -jnp.sum(l2), jnp.float32)

    @pl.when(pl.program_id(0) == 0)
    def _init():
        loss_ref[...] = jnp.zeros((1, _D), jnp.float32)

    loss_ref[...] += blk_loss


def _mlp(eu, ep, en, w0u, w0i, b0r, w1p, b1p, w2p, b2p, wpp, bp11, batch):
    n_blk = batch // _BLK
    row_spec = pl.BlockSpec((_BLK, _D), lambda i: (i, 0))
    w_spec = pl.BlockSpec((_D, _D), lambda i: (0, 0))
    c_spec = pl.BlockSpec((_D, 1), lambda i: (0, 0))
    return pl.pallas_call(
        _mlp_body,
        grid=(n_blk,),
        in_specs=[row_spec, row_spec, row_spec,
                  w_spec, w_spec, c_spec,
                  w_spec, c_spec,
                  w_spec, c_spec,
                  c_spec,
                  pl.BlockSpec(memory_space=pltpu.SMEM)],
        out_specs=[pl.BlockSpec((1, 1, _BLK), lambda i: (i, 0, 0)),
                   pl.BlockSpec((1, 1, _BLK), lambda i: (i, 0, 0)),
                   pl.BlockSpec((1, _D), lambda i: (0, 0))],
        out_shape=[jax.ShapeDtypeStruct((n_blk, 1, _BLK), jnp.float32),
                   jax.ShapeDtypeStruct((n_blk, 1, _BLK), jnp.float32),
                   jax.ShapeDtypeStruct((1, _D), jnp.float32)],
    )(eu, ep, en, w0u, w0i, b0r, w1p, b1p, w2p, b2p, wpp, bp11)


_NCHUNK = 1


def kernel(user, pos_item, neg_item, user_table, item_table,
           W0, b0, W1, b1, W2, b2, Wp, bp):
    batch = user.shape[0]
    user2d = user.astype(jnp.int32).reshape(batch // _CH, _CH)
    pos2d = pos_item.astype(jnp.int32).reshape(batch // _CH, _CH)
    neg2d = neg_item.astype(jnp.int32).reshape(batch // _CH, _CH)

    w0u = W0[:_D].T
    w0i = W0[_D:].T
    b0r = b0.reshape(_D, 1)
    w1p = jnp.zeros((_D, _D), jnp.float32).at[:64, :].set(W1.T)
    b1p = jnp.zeros((_D, 1), jnp.float32).at[:64, 0].set(b1)
    w2p = jnp.zeros((_D, _D), jnp.float32).at[:32, :64].set(W2.T)
    b2p = jnp.zeros((_D, 1), jnp.float32).at[:32, 0].set(b2)
    wpp = jnp.zeros((_D, 1), jnp.float32).at[:32, 0].set(Wp[:, 0])
    bp11 = bp.reshape(1, 1)

    rows = batch // _NCHUNK
    irows = rows // _CH
    pps, pns, losses = [], [], []
    for c in range(_NCHUNK):
        sl = slice(c * irows, (c + 1) * irows)
        eu, ep, en = _gather3(user2d[sl], pos2d[sl], neg2d[sl],
                              user_table, item_table, rows)
        pp, pn, ls = _mlp(eu, ep, en, w0u, w0i, b0r, w1p, b1p, w2p, b2p,
                          wpp, bp11, rows)
        pps.append(pp.reshape(rows))
        pns.append(pn.reshape(rows))
        losses.append(ls)
    pp = jnp.concatenate(pps) if _NCHUNK > 1 else pps[0]
    pn = jnp.concatenate(pns) if _NCHUNK > 1 else pns[0]
    loss = sum(losses[1:], losses[0])
    return pp, pn, loss[0, 0].reshape(())


# speedup vs baseline: 1.0331x; 1.0074x over previous
import functools

import jax
import jax.numpy as jnp
from jax import lax
from jax.experimental import pallas as pl
from jax.experimental.pallas import tpu as pltpu
from jax.experimental.pallas import tpu_sc as plsc

_D = 128
_CH = 128
_BLK = 4096
_INV_LN2 = 1.4426950408889634


def _gather3(user2d, pos2d, neg2d, user_table, item_table, batch):
    info = plsc.get_sparse_core_info()
    nw = info.num_cores * info.num_subcores
    rows_per_w = batch // nw
    cpg = rows_per_w // _CH
    ntask = 3 * cpg
    mesh = plsc.VectorSubcoreMesh(core_axis_name="c", subcore_axis_name="s")

    @functools.partial(
        pl.kernel,
        mesh=mesh,
        out_type=(jax.ShapeDtypeStruct((batch, _D), jnp.float32),) * 3,
        scratch_types=[
            pltpu.VMEM((ntask, _CH), jnp.int32),
            pltpu.VMEM((4, _CH, _D), jnp.float32),
            pltpu.SemaphoreType.DMA,
            pltpu.SemaphoreType.DMA,
            pltpu.SemaphoreType.DMA,
            pltpu.SemaphoreType.DMA,
            pltpu.SemaphoreType.DMA,
            pltpu.SemaphoreType.DMA,
            pltpu.SemaphoreType.DMA,
            pltpu.SemaphoreType.DMA,
            pltpu.SemaphoreType.DMA,
        ],
    )
    def k(user_h, pos_h, neg_h, ut_h, it_h, eu_h, ep_h, en_h,
          idx_v, rows_v, g0, g1, g2, g3, s0, s1, s2, s3, isem):
        wid = lax.axis_index("s") * info.num_cores + lax.axis_index("c")
        rbase = wid * rows_per_w
        irow0 = wid * cpg
        ic = [pltpu.async_copy(src.at[pl.ds(irow0, cpg)],
                               idx_v.at[pl.ds(off * cpg, cpg)], isem)
              for off, src in enumerate((user_h, pos_h, neg_h))]
        for c in ic:
            c.wait()

        gsem = (g0, g1, g2, g3)
        ssem = (s0, s1, s2, s3)
        tasks = []
        for j, (tbl, out) in enumerate(((ut_h, eu_h), (it_h, ep_h), (it_h, en_h))):
            for c in range(cpg):
                tasks.append((j * cpg + c, tbl, out, rbase + c * _CH))

        scat = [None, None, None, None]
        q = []

        def drain_one():
            pg, pb, pout, pobase = q.pop(0)
            pg.wait()
            scat[pb] = pltpu.async_copy(
                rows_v.at[pb], pout.at[pl.ds(pobase, _CH)], ssem[pb])

        for t, (ti, tbl, out, obase) in enumerate(tasks):
            b = t % 4
            if scat[b] is not None:
                scat[b].wait()
                scat[b] = None
            q.append((pltpu.async_copy(tbl.at[idx_v.at[ti]], rows_v.at[b],
                                       gsem[b]), b, out, obase))
            if len(q) >= 3:
                drain_one()
        while q:
            drain_one()
        for sc in scat:
            if sc is not None:
                sc.wait()

    return k(user2d, pos2d, neg2d, user_table, item_table)


_RT = (((1,), (1,)), ((), ()))


def _mlp_body(eu_ref, ep_ref, en_ref, w0u_ref, w0i_ref, b0_ref,
              w1_ref, b1_ref, w2_ref, b2_ref, wp_ref, bp_ref,
              pp_ref, pn_ref, loss_ref):
    bf = jnp.bfloat16
    eu = eu_ref[...].astype(bf)
    ep = ep_ref[...].astype(bf)
    en = en_ref[...].astype(bf)
    w0u = w0u_ref[...].astype(bf)
    w0i = w0i_ref[...].astype(bf)
    aT = lax.dot_general(w0u, eu, _RT, preferred_element_type=jnp.float32)
    b0 = b0_ref[...]
    h0p = jnp.maximum(
        aT + lax.dot_general(w0i, ep, _RT,
                             preferred_element_type=jnp.float32) + b0, 0.0)
    h0n = jnp.maximum(
        aT + lax.dot_general(w0i, en, _RT,
                             preferred_element_type=jnp.float32) + b0, 0.0)
    w1 = w1_ref[...].astype(bf)
    b1 = b1_ref[...]
    h1p = jnp.maximum(
        jnp.dot(w1, h0p.astype(bf), preferred_element_type=jnp.float32)
        + b1, 0.0)
    h1n = jnp.maximum(
        jnp.dot(w1, h0n.astype(bf), preferred_element_type=jnp.float32)
        + b1, 0.0)
    w2 = w2_ref[...].astype(bf)
    b2 = b2_ref[...]
    h2p = jnp.maximum(
        jnp.dot(w2, h1p.astype(bf), preferred_element_type=jnp.float32)
        + b2, 0.0)
    h2n = jnp.maximum(
        jnp.dot(w2, h1n.astype(bf), preferred_element_type=jnp.float32)
        + b2, 0.0)
    wp = wp_ref[...]
    bp = bp_ref[0, 0]
    pp = jnp.sum(h2p * wp, axis=0, keepdims=True) + bp
    pn = jnp.sum(h2n * wp, axis=0, keepdims=True) + bp
    pp_ref[...] = pp[None]
    pn_ref[...] = pn[None]
    d = pp - pn
    l2 = -(jnp.maximum(-d, 0.0)
           + jnp.log(1.0 + jnp.exp(-jnp.abs(d)))) * _INV_LN2
    blk_loss = jnp.full((1, _D), -jnp.sum(l2), jnp.float32)

    @pl.when(pl.program_id(0) == 0)
    def _init():
        loss_ref[...] = jnp.zeros((1, _D), jnp.float32)

    loss_ref[...] += blk_loss


def _mlp(eu, ep, en, w0u, w0i, b0r, w1p, b1p, w2p, b2p, wpp, bp11, batch):
    n_blk = batch // _BLK
    row_spec = pl.BlockSpec((_BLK, _D), lambda i: (i, 0))
    w_spec = pl.BlockSpec((_D, _D), lambda i: (0, 0))
    c_spec = pl.BlockSpec((_D, 1), lambda i: (0, 0))
    return pl.pallas_call(
        _mlp_body,
        grid=(n_blk,),
        in_specs=[row_spec, row_spec, row_spec,
                  w_spec, w_spec, c_spec,
                  w_spec, c_spec,
                  w_spec, c_spec,
                  c_spec,
                  pl.BlockSpec(memory_space=pltpu.SMEM)],
        out_specs=[pl.BlockSpec((1, 1, _BLK), lambda i: (i, 0, 0)),
                   pl.BlockSpec((1, 1, _BLK), lambda i: (i, 0, 0)),
                   pl.BlockSpec((1, _D), lambda i: (0, 0))],
        out_shape=[jax.ShapeDtypeStruct((n_blk, 1, _BLK), jnp.float32),
                   jax.ShapeDtypeStruct((n_blk, 1, _BLK), jnp.float32),
                   jax.ShapeDtypeStruct((1, _D), jnp.float32)],
    )(eu, ep, en, w0u, w0i, b0r, w1p, b1p, w2p, b2p, wpp, bp11)


_NCHUNK = 1


def kernel(user, pos_item, neg_item, user_table, item_table,
           W0, b0, W1, b1, W2, b2, Wp, bp):
    batch = user.shape[0]
    user2d = user.astype(jnp.int32).reshape(batch // _CH, _CH)
    pos2d = pos_item.astype(jnp.int32).reshape(batch // _CH, _CH)
    neg2d = neg_item.astype(jnp.int32).reshape(batch // _CH, _CH)

    w0u = W0[:_D].T
    w0i = W0[_D:].T
    b0r = b0.reshape(_D, 1)
    w1p = jnp.zeros((_D, _D), jnp.float32).at[:64, :].set(W1.T)
    b1p = jnp.zeros((_D, 1), jnp.float32).at[:64, 0].set(b1)
    w2p = jnp.zeros((_D, _D), jnp.float32).at[:32, :64].set(W2.T)
    b2p = jnp.zeros((_D, 1), jnp.float32).at[:32, 0].set(b2)
    wpp = jnp.zeros((_D, 1), jnp.float32).at[:32, 0].set(Wp[:, 0])
    bp11 = bp.reshape(1, 1)

    rows = batch // _NCHUNK
    irows = rows // _CH
    pps, pns, losses = [], [], []
    for c in range(_NCHUNK):
        sl = slice(c * irows, (c + 1) * irows)
        eu, ep, en = _gather3(user2d[sl], pos2d[sl], neg2d[sl],
                              user_table, item_table, rows)
        pp, pn, ls = _mlp(eu, ep, en, w0u, w0i, b0r, w1p, b1p, w2p, b2p,
                          wpp, bp11, rows)
        pps.append(pp.reshape(rows))
        pns.append(pn.reshape(rows))
        losses.append(ls)
    pp = jnp.concatenate(pps) if _NCHUNK > 1 else pps[0]
    pn = jnp.concatenate(pns) if _NCHUNK > 1 else pns[0]
    loss = sum(losses[1:], losses[0])
    return pp, pn, loss[0, 0].reshape(())


# speedup vs baseline: 1.0472x; 1.0136x over previous
import functools

import jax
import jax.numpy as jnp
from jax import lax
from jax.experimental import pallas as pl
from jax.experimental.pallas import tpu as pltpu
from jax.experimental.pallas import tpu_sc as plsc

_D = 128
_CH = 128
_BLK = 4096
_INV_LN2 = 1.4426950408889634


def _gather3(user2d, pos2d, neg2d, user_table, item_table, batch):
    info = plsc.get_sparse_core_info()
    nw = info.num_cores * info.num_subcores
    rows_per_w = batch // nw
    cpg = rows_per_w // _CH
    ntask = 3 * cpg
    mesh = plsc.VectorSubcoreMesh(core_axis_name="c", subcore_axis_name="s")

    @functools.partial(
        pl.kernel,
        mesh=mesh,
        out_type=(jax.ShapeDtypeStruct((batch, _D), jnp.float32),) * 3,
        scratch_types=[
            pltpu.VMEM((ntask, _CH), jnp.int32),
            pltpu.VMEM((6, _CH, _D), jnp.float32),
        ] + [pltpu.SemaphoreType.DMA] * 13,
    )
    def k(user_h, pos_h, neg_h, ut_h, it_h, eu_h, ep_h, en_h,
          idx_v, rows_v, *sems):
        wid = lax.axis_index("s") * info.num_cores + lax.axis_index("c")
        rbase = wid * rows_per_w
        irow0 = wid * cpg
        gsem = sems[0:6]
        ssem = sems[6:12]
        isem = sems[12]
        ic = [pltpu.async_copy(src.at[pl.ds(irow0, cpg)],
                               idx_v.at[pl.ds(off * cpg, cpg)], isem)
              for off, src in enumerate((user_h, pos_h, neg_h))]

        tasks = []
        for j, (tbl, out) in enumerate(((ut_h, eu_h), (it_h, ep_h), (it_h, en_h))):
            for c in range(cpg):
                tasks.append((j, j * cpg + c, tbl, out, rbase + c * _CH))

        scat = [None] * 6
        q = []

        def drain_one():
            pg, pb, pout, pobase = q.pop(0)
            pg.wait()
            scat[pb] = pltpu.async_copy(
                rows_v.at[pb], pout.at[pl.ds(pobase, _CH)], ssem[pb])

        for t, (tj, ti, tbl, out, obase) in enumerate(tasks):
            if ic[tj] is not None:
                ic[tj].wait()
                ic[tj] = None
            b = t % 6
            if scat[b] is not None:
                scat[b].wait()
                scat[b] = None
            q.append((pltpu.async_copy(tbl.at[idx_v.at[ti]], rows_v.at[b],
                                       gsem[b]), b, out, obase))
            if len(q) >= 4:
                drain_one()
        while q:
            drain_one()
        for sc in scat:
            if sc is not None:
                sc.wait()

    return k(user2d, pos2d, neg2d, user_table, item_table)


_RT = (((1,), (1,)), ((), ()))


def _mlp_body(eu_ref, ep_ref, en_ref, w0u_ref, w0i_ref, b0_ref,
              w1_ref, b1_ref, w2_ref, b2_ref, wp_ref, bp_ref,
              pp_ref, pn_ref, loss_ref):
    eu = eu_ref[...]
    ep = ep_ref[...]
    en = en_ref[...]
    aT = lax.dot_general(w0u_ref[...], eu, _RT,
                         preferred_element_type=jnp.float32)
    b0 = b0_ref[...]
    h0p = jnp.maximum(
        aT + lax.dot_general(w0i_ref[...], ep, _RT,
                             preferred_element_type=jnp.float32) + b0, 0.0)
    h0n = jnp.maximum(
        aT + lax.dot_general(w0i_ref[...], en, _RT,
                             preferred_element_type=jnp.float32) + b0, 0.0)
    w1 = w1_ref[...]
    b1 = b1_ref[...]
    h1p = jnp.maximum(
        jnp.dot(w1, h0p, preferred_element_type=jnp.float32) + b1, 0.0)
    h1n = jnp.maximum(
        jnp.dot(w1, h0n, preferred_element_type=jnp.float32) + b1, 0.0)
    w2 = w2_ref[...]
    b2 = b2_ref[...]
    h2p = jnp.maximum(
        jnp.dot(w2, h1p, preferred_element_type=jnp.float32) + b2, 0.0)
    h2n = jnp.maximum(
        jnp.dot(w2, h1n, preferred_element_type=jnp.float32) + b2, 0.0)
    wp = wp_ref[...]
    bp = bp_ref[0, 0]
    pp = jnp.sum(h2p * wp, axis=0, keepdims=True) + bp
    pn = jnp.sum(h2n * wp, axis=0, keepdims=True) + bp
    pp_ref[...] = pp[None]
    pn_ref[...] = pn[None]
    d = pp - pn
    l2 = -(jnp.maximum(-d, 0.0)
           + jnp.log(1.0 + jnp.exp(-jnp.abs(d)))) * _INV_LN2
    blk_loss = jnp.full((1, _D), -jnp.sum(l2), jnp.float32)

    @pl.when(pl.program_id(0) == 0)
    def _init():
        loss_ref[...] = jnp.zeros((1, _D), jnp.float32)

    loss_ref[...] += blk_loss


def _mlp(eu, ep, en, w0u, w0i, b0r, w1p, b1p, w2p, b2p, wpp, bp11, batch):
    n_blk = batch // _BLK
    row_spec = pl.BlockSpec((_BLK, _D), lambda i: (i, 0))
    w_spec = pl.BlockSpec((_D, _D), lambda i: (0, 0))
    c_spec = pl.BlockSpec((_D, 1), lambda i: (0, 0))
    return pl.pallas_call(
        _mlp_body,
        grid=(n_blk,),
        in_specs=[row_spec, row_spec, row_spec,
                  w_spec, w_spec, c_spec,
                  w_spec, c_spec,
                  w_spec, c_spec,
                  c_spec,
                  pl.BlockSpec(memory_space=pltpu.SMEM)],
        out_specs=[pl.BlockSpec((1, 1, _BLK), lambda i: (i, 0, 0)),
                   pl.BlockSpec((1, 1, _BLK), lambda i: (i, 0, 0)),
                   pl.BlockSpec((1, _D), lambda i: (0, 0))],
        out_shape=[jax.ShapeDtypeStruct((n_blk, 1, _BLK), jnp.float32),
                   jax.ShapeDtypeStruct((n_blk, 1, _BLK), jnp.float32),
                   jax.ShapeDtypeStruct((1, _D), jnp.float32)],
    )(eu, ep, en, w0u, w0i, b0r, w1p, b1p, w2p, b2p, wpp, bp11)


_NCHUNK = 1


def kernel(user, pos_item, neg_item, user_table, item_table,
           W0, b0, W1, b1, W2, b2, Wp, bp):
    batch = user.shape[0]
    user2d = user.astype(jnp.int32).reshape(batch // _CH, _CH)
    pos2d = pos_item.astype(jnp.int32).reshape(batch // _CH, _CH)
    neg2d = neg_item.astype(jnp.int32).reshape(batch // _CH, _CH)

    w0u = W0[:_D].T
    w0i = W0[_D:].T
    b0r = b0.reshape(_D, 1)
    w1p = jnp.zeros((_D, _D), jnp.float32).at[:64, :].set(W1.T)
    b1p = jnp.zeros((_D, 1), jnp.float32).at[:64, 0].set(b1)
    w2p = jnp.zeros((_D, _D), jnp.float32).at[:32, :64].set(W2.T)
    b2p = jnp.zeros((_D, 1), jnp.float32).at[:32, 0].set(b2)
    wpp = jnp.zeros((_D, 1), jnp.float32).at[:32, 0].set(Wp[:, 0])
    bp11 = bp.reshape(1, 1)

    rows = batch // _NCHUNK
    irows = rows // _CH
    pps, pns, losses = [], [], []
    for c in range(_NCHUNK):
        sl = slice(c * irows, (c + 1) * irows)
        eu, ep, en = _gather3(user2d[sl], pos2d[sl], neg2d[sl],
                              user_table, item_table, rows)
        pp, pn, ls = _mlp(eu, ep, en, w0u, w0i, b0r, w1p, b1p, w2p, b2p,
                          wpp, bp11, rows)
        pps.append(pp.reshape(rows))
        pns.append(pn.reshape(rows))
        losses.append(ls)
    pp = jnp.concatenate(pps) if _NCHUNK > 1 else pps[0]
    pn = jnp.concatenate(pns) if _NCHUNK > 1 else pns[0]
    loss = sum(losses[1:], losses[0])
    return pp, pn, loss[0, 0].reshape(())
